# baseline (speedup 1.0000x reference)
import functools

import jax
import jax.numpy as jnp
from jax import lax
from jax.experimental import pallas as pl
from jax.experimental.pallas import tpu as pltpu
from jax.experimental.pallas import tpu_sc as plsc

N_NODES = 10000
N_EDGES = 320000
D = 128
NC = 2
NS = 16
NW = NC * NS
CH = 80
CPT = 125
E_PAD = NW * CPT * CH
N_PAD = 10240
ROWS_PT = N_PAD // NS
NBUF = 4
SCALE = 256.0


def _sc_aggregate(Xq, idxA, idxB, zeros):
    mesh = plsc.VectorSubcoreMesh(core_axis_name="c", subcore_axis_name="s")

    @functools.partial(
        pl.kernel,
        out_type=jax.ShapeDtypeStruct((NC, N_PAD, D), jnp.int16),
        mesh=mesh,
        scratch_types=[
            pltpu.VMEM_SHARED((N_PAD, D), jnp.int16),
            pltpu.VMEM((CPT, CH), jnp.int32),
            pltpu.VMEM((CPT, CH), jnp.int32),
            [pltpu.VMEM((CH, D), jnp.int16) for _ in range(NBUF)],
            [pltpu.SemaphoreType.DMA for _ in range(NBUF)],
            [pltpu.SemaphoreType.DMA for _ in range(NBUF)],
        ],
        compiler_params=pltpu.CompilerParams(use_tc_tiling_on_sc=False),
    )
    def agg_kernel(x_hbm, ia_hbm, ib_hbm, z_hbm, out_hbm,
                   acc_sh, ia_v, ib_v, bufs, gsems, ssems):
        c = lax.axis_index("c")
        s = lax.axis_index("s")
        w = c * NS + s

        pltpu.sync_copy(z_hbm, bufs[0])
        for r in range(ROWS_PT // CH):
            pltpu.sync_copy(bufs[0], acc_sh.at[pl.ds(s * ROWS_PT + r * CH, CH)])

        pltpu.sync_copy(ia_hbm.at[w], ia_v)
        pltpu.sync_copy(ib_hbm.at[w], ib_v)

        plsc.subcore_barrier()

        def start_gather(b, cix):
            pltpu.async_copy(x_hbm.at[ia_v.at[cix]], bufs[b], gsems[b])

        def wait_gather(b, cix):
            pltpu.make_async_copy(x_hbm.at[ia_v.at[cix]], bufs[b], gsems[b]).wait()

        def start_scatter(b, cix):
            pltpu.async_copy(bufs[b], acc_sh.at[ib_v.at[cix]], ssems[b], add=True)

        def wait_scatter(b, cix):
            pltpu.make_async_copy(bufs[b], acc_sh.at[ib_v.at[cix]], ssems[b]).wait()

        start_gather(0, 0)
        start_gather(1, 1)

        @pl.loop(0, CPT // NBUF)
        def _(j):
            for b in range(NBUF):
                cix = j * NBUF + b
                b2 = (b + 2) % NBUF
                wait_gather(b, cix)
                start_scatter(b, cix)

                @pl.when(cix >= 2)
                def _():
                    wait_scatter(b2, cix - 2)

                @pl.when(cix + 2 < CPT)
                def _():
                    start_gather(b2, cix + 2)

        for k in range((CPT // NBUF) * NBUF, CPT):
            b = k % NBUF
            wait_gather(b, k)
            start_scatter(b, k)
            wait_scatter((b + 2) % NBUF, k - 2)

        wait_scatter((CPT - 2) % NBUF, CPT - 2)
        wait_scatter((CPT - 1) % NBUF, CPT - 1)

        plsc.subcore_barrier()

        pltpu.sync_copy(acc_sh.at[pl.ds(s * ROWS_PT, ROWS_PT)],
                        out_hbm.at[c, pl.ds(s * ROWS_PT, ROWS_PT)])

    return agg_kernel(Xq, idxA, idxB, zeros)


def _mlp_body(x_ref, p_hbm, w1_ref, b1_ref, w2_ref, b2_ref, o_ref,
              p_vmem, dsem):
    i = pl.program_id(0)
    pltpu.make_async_copy(
        p_hbm.at[:, pl.ds(i * x_ref.shape[0], x_ref.shape[0])], p_vmem, dsem
    ).start()
    pltpu.make_async_copy(
        p_hbm.at[:, pl.ds(i * x_ref.shape[0], x_ref.shape[0])], p_vmem, dsem
    ).wait()
    agg = (p_vmem[0].astype(jnp.float32)
           + p_vmem[1].astype(jnp.float32)) * jnp.float32(1.0 / SCALE)
    conv = x_ref[...] + agg
    h = jnp.maximum(
        jnp.dot(conv, w1_ref[...], preferred_element_type=jnp.float32)
        + b1_ref[...], 0.0)
    o_ref[...] = jnp.maximum(
        jnp.dot(h, w2_ref[...], preferred_element_type=jnp.float32)
        + b2_ref[...], 0.0)


def _tc_mlp(X, partials, w1, b1, w2, b2):
    BR = 1000
    return pl.pallas_call(
        _mlp_body,
        grid=(N_NODES // BR,),
        in_specs=[
            pl.BlockSpec((BR, D), lambda i: (i, 0)),
            pl.BlockSpec(memory_space=pl.ANY),
            pl.BlockSpec((D, D), lambda i: (0, 0)),
            pl.BlockSpec((D,), lambda i: (0,)),
            pl.BlockSpec((D, D), lambda i: (0, 0)),
            pl.BlockSpec((D,), lambda i: (0,)),
        ],
        out_specs=pl.BlockSpec((BR, D), lambda i: (i, 0)),
        out_shape=jax.ShapeDtypeStruct((N_NODES, D), jnp.float32),
        scratch_shapes=[
            pltpu.VMEM((NC, BR, D), jnp.int16),
            pltpu.SemaphoreType.DMA,
        ],
    )(X, partials, w1, b1, w2, b2)


def kernel(X, ref_A, ref_B, w1, b1, w2, b2):
    npad = E_PAD - N_EDGES
    idxA = jnp.concatenate(
        [ref_A.astype(jnp.int32), jnp.zeros((npad,), jnp.int32)]
    ).reshape(NW, CPT, CH)
    idxB = jnp.concatenate(
        [ref_B.astype(jnp.int32), jnp.full((npad,), N_PAD - 1, jnp.int32)]
    ).reshape(NW, CPT, CH)
    Xq = jnp.round(X * SCALE).astype(jnp.int16)
    zeros = jnp.zeros((CH, D), jnp.int16)
    partials = _sc_aggregate(Xq, idxA, idxB, zeros)
    return _tc_mlp(X, partials, w1, b1, w2, b2)


# speedup vs baseline: 1.0935x; 1.0935x over previous
import functools

import jax
import jax.numpy as jnp
from jax import lax
from jax.experimental import pallas as pl
from jax.experimental.pallas import tpu as pltpu
from jax.experimental.pallas import tpu_sc as plsc

N_NODES = 10000
N_EDGES = 320000
D = 128
NC = 2
NS = 16
NW = NC * NS
CH = 80
CPT = N_EDGES // NW // CH
ROWS_PT = N_NODES // NS
NBUF = 3


def _sc_aggregate(X, idxA, idxB, zeros):
    mesh = plsc.VectorSubcoreMesh(core_axis_name="c", subcore_axis_name="s")

    @functools.partial(
        pl.kernel,
        out_type=jax.ShapeDtypeStruct((NC, N_NODES, D), jnp.float32),
        mesh=mesh,
        scratch_types=[
            pltpu.VMEM_SHARED((N_NODES, D), jnp.float32),
            pltpu.VMEM((CPT, CH), jnp.int32),
            pltpu.VMEM((CPT, CH), jnp.int32),
            [pltpu.VMEM((CH, D), jnp.float32) for _ in range(NBUF)],
            [pltpu.SemaphoreType.DMA for _ in range(NBUF)],
            [pltpu.SemaphoreType.DMA for _ in range(NBUF)],
        ],
        compiler_params=pltpu.CompilerParams(use_tc_tiling_on_sc=False),
    )
    def agg_kernel(x_hbm, ia_hbm, ib_hbm, z_hbm, out_hbm,
                   acc_sh, ia_v, ib_v, bufs, gsems, ssems):
        c = lax.axis_index("c")
        s = lax.axis_index("s")
        w = c * NS + s

        pltpu.sync_copy(z_hbm, bufs[0])
        for r in range(ROWS_PT // CH):
            pltpu.sync_copy(bufs[0], acc_sh.at[pl.ds(s * ROWS_PT + r * CH, CH)])
        rem = ROWS_PT % CH
        if rem:
            pltpu.sync_copy(
                bufs[0].at[pl.ds(0, rem)],
                acc_sh.at[pl.ds(s * ROWS_PT + (ROWS_PT // CH) * CH, rem)])

        pltpu.sync_copy(ia_hbm.at[w], ia_v)
        pltpu.sync_copy(ib_hbm.at[w], ib_v)

        plsc.subcore_barrier()

        def start_gather(b, cix):
            pltpu.async_copy(x_hbm.at[ia_v.at[cix]], bufs[b], gsems[b])

        def wait_gather(b, cix):
            pltpu.make_async_copy(x_hbm.at[ia_v.at[cix]], bufs[b], gsems[b]).wait()

        def start_scatter(b, cix):
            pltpu.async_copy(bufs[b], acc_sh.at[ib_v.at[cix]], ssems[b], add=True)

        def wait_scatter(b, cix):
            pltpu.make_async_copy(bufs[b], acc_sh.at[ib_v.at[cix]], ssems[b]).wait()

        start_gather(0, 0)
        start_gather(1, 1)

        def visit(cix, b, guarded):
            b2 = (b + 2) % NBUF
            wait_gather(b, cix)
            start_scatter(b, cix)
            if guarded:
                @pl.when(cix >= 1)
                def _():
                    wait_scatter(b2, cix - 1)

                @pl.when(cix + 2 < CPT)
                def _():
                    start_gather(b2, cix + 2)
            else:
                wait_scatter(b2, cix - 1)
                if cix + 2 < CPT:
                    start_gather(b2, cix + 2)

        NFULL = (CPT // NBUF) * NBUF
        @pl.loop(0, CPT // NBUF)
        def _(j):
            for b in range(NBUF):
                visit(j * NBUF + b, b, True)

        for cix in range(NFULL, CPT):
            visit(cix, cix % NBUF, False)

        wait_scatter((CPT - 1) % NBUF, CPT - 1)

        plsc.subcore_barrier()

        pltpu.sync_copy(acc_sh.at[pl.ds(s * ROWS_PT, ROWS_PT)],
                        out_hbm.at[c, pl.ds(s * ROWS_PT, ROWS_PT)])

    return agg_kernel(X, idxA, idxB, zeros)


def _mlp_body(x_ref, p_ref, w1_ref, b1_ref, w2_ref, b2_ref, o_ref):
    conv = x_ref[...] + p_ref[0] + p_ref[1]
    h = jnp.maximum(
        jnp.dot(conv, w1_ref[...], preferred_element_type=jnp.float32)
        + b1_ref[...], 0.0)
    o_ref[...] = jnp.maximum(
        jnp.dot(h, w2_ref[...], preferred_element_type=jnp.float32)
        + b2_ref[...], 0.0)


def _tc_mlp(X, partials, w1, b1, w2, b2):
    BR = 1000
    return pl.pallas_call(
        _mlp_body,
        grid=(N_NODES // BR,),
        in_specs=[
            pl.BlockSpec((BR, D), lambda i: (i, 0)),
            pl.BlockSpec((NC, BR, D), lambda i: (0, i, 0)),
            pl.BlockSpec((D, D), lambda i: (0, 0)),
            pl.BlockSpec((D,), lambda i: (0,)),
            pl.BlockSpec((D, D), lambda i: (0, 0)),
            pl.BlockSpec((D,), lambda i: (0,)),
        ],
        out_specs=pl.BlockSpec((BR, D), lambda i: (i, 0)),
        out_shape=jax.ShapeDtypeStruct((N_NODES, D), jnp.float32),
    )(X, partials, w1, b1, w2, b2)


def kernel(X, ref_A, ref_B, w1, b1, w2, b2):
    idxA = ref_A.astype(jnp.int32).reshape(NW, CPT, CH)
    idxB = ref_B.astype(jnp.int32).reshape(NW, CPT, CH)
    zeros = jnp.zeros((CH, D), jnp.float32)
    partials = _sc_aggregate(X, idxA, idxB, zeros)
    return _tc_mlp(X, partials, w1, b1, w2, b2)


# speedup vs baseline: 1.1255x; 1.0293x over previous
import functools

import jax
import jax.numpy as jnp
from jax import lax
from jax.experimental import pallas as pl
from jax.experimental.pallas import tpu as pltpu
from jax.experimental.pallas import tpu_sc as plsc

N_NODES = 10000
N_EDGES = 320000
D = 128
NC = 2
NS = 16
NW = NC * NS
CH = 80
CPT = N_EDGES // NW // CH
ROWS_PT = N_NODES // NS
NBUF = 3


def _sc_aggregate(X, idxA, idxB, zeros):
    mesh = plsc.VectorSubcoreMesh(core_axis_name="c", subcore_axis_name="s")

    @functools.partial(
        pl.kernel,
        out_type=jax.ShapeDtypeStruct((NC, N_NODES, D), jnp.float32),
        mesh=mesh,
        scratch_types=[
            pltpu.VMEM_SHARED((N_NODES, D), jnp.float32),
            pltpu.VMEM((CPT, CH), jnp.int32),
            pltpu.VMEM((CPT, CH), jnp.int32),
            [pltpu.VMEM((CH, D), jnp.float32) for _ in range(NBUF)],
            [pltpu.SemaphoreType.DMA for _ in range(NBUF)],
            [pltpu.SemaphoreType.DMA for _ in range(NBUF)],
        ],
        compiler_params=pltpu.CompilerParams(use_tc_tiling_on_sc=False),
    )
    def agg_kernel(x_hbm, ia_hbm, ib_hbm, z_hbm, out_hbm,
                   acc_sh, ia_v, ib_v, bufs, gsems, ssems):
        c = lax.axis_index("c")
        s = lax.axis_index("s")
        w = c * NS + s

        pltpu.sync_copy(ia_hbm.at[w], ia_v)
        pltpu.sync_copy(ib_hbm.at[w], ib_v)

        def start_gather(b, cix):
            pltpu.async_copy(x_hbm.at[ia_v.at[cix]], bufs[b], gsems[b])

        def wait_gather(b, cix):
            pltpu.make_async_copy(x_hbm.at[ia_v.at[cix]], bufs[b], gsems[b]).wait()

        def start_scatter(b, cix):
            pltpu.async_copy(bufs[b], acc_sh.at[ib_v.at[cix]], ssems[b], add=True)

        def wait_scatter(b, cix):
            pltpu.make_async_copy(bufs[b], acc_sh.at[ib_v.at[cix]], ssems[b]).wait()

        start_gather(0, 0)
        start_gather(1, 1)

        pltpu.sync_copy(z_hbm, bufs[2])
        for r in range(ROWS_PT // CH):
            pltpu.sync_copy(bufs[2], acc_sh.at[pl.ds(s * ROWS_PT + r * CH, CH)])
        rem = ROWS_PT % CH
        if rem:
            pltpu.sync_copy(
                bufs[2].at[pl.ds(0, rem)],
                acc_sh.at[pl.ds(s * ROWS_PT + (ROWS_PT // CH) * CH, rem)])

        plsc.subcore_barrier()

        def visit(cix, b, guarded):
            b2 = (b + 2) % NBUF
            wait_gather(b, cix)
            start_scatter(b, cix)
            if guarded:
                @pl.when(cix >= 1)
                def _():
                    wait_scatter(b2, cix - 1)

                @pl.when(cix + 2 < CPT)
                def _():
                    start_gather(b2, cix + 2)
            else:
                wait_scatter(b2, cix - 1)
                if cix + 2 < CPT:
                    start_gather(b2, cix + 2)

        NFULL = (CPT // NBUF) * NBUF
        @pl.loop(0, CPT // NBUF)
        def _(j):
            for b in range(NBUF):
                visit(j * NBUF + b, b, True)

        for cix in range(NFULL, CPT):
            visit(cix, cix % NBUF, False)

        wait_scatter((CPT - 1) % NBUF, CPT - 1)

        plsc.subcore_barrier()

        pltpu.sync_copy(acc_sh.at[pl.ds(s * ROWS_PT, ROWS_PT)],
                        out_hbm.at[c, pl.ds(s * ROWS_PT, ROWS_PT)])

    return agg_kernel(X, idxA, idxB, zeros)


def _mlp_body(x_ref, p_ref, w1_ref, b1_ref, w2_ref, b2_ref, o_ref):
    conv = x_ref[...] + p_ref[0] + p_ref[1]
    h = jnp.maximum(
        jnp.dot(conv, w1_ref[...], preferred_element_type=jnp.float32)
        + b1_ref[...], 0.0)
    o_ref[...] = jnp.maximum(
        jnp.dot(h, w2_ref[...], preferred_element_type=jnp.float32)
        + b2_ref[...], 0.0)


def _tc_mlp(X, partials, w1, b1, w2, b2):
    BR = 2000
    return pl.pallas_call(
        _mlp_body,
        grid=(N_NODES // BR,),
        in_specs=[
            pl.BlockSpec((BR, D), lambda i: (i, 0)),
            pl.BlockSpec((NC, BR, D), lambda i: (0, i, 0)),
            pl.BlockSpec((D, D), lambda i: (0, 0)),
            pl.BlockSpec((D,), lambda i: (0,)),
            pl.BlockSpec((D, D), lambda i: (0, 0)),
            pl.BlockSpec((D,), lambda i: (0,)),
        ],
        out_specs=pl.BlockSpec((BR, D), lambda i: (i, 0)),
        out_shape=jax.ShapeDtypeStruct((N_NODES, D), jnp.float32),
    )(X, partials, w1, b1, w2, b2)


def kernel(X, ref_A, ref_B, w1, b1, w2, b2):
    idxA = ref_A.astype(jnp.int32).reshape(NW, CPT, CH)
    idxB = ref_B.astype(jnp.int32).reshape(NW, CPT, CH)
    zeros = jnp.zeros((CH, D), jnp.float32)
    partials = _sc_aggregate(X, idxA, idxB, zeros)
    return _tc_mlp(X, partials, w1, b1, w2, b2)
